# baseline (speedup 1.0000x reference)
import functools

import jax
import jax.numpy as jnp
from jax import lax
from jax.experimental import pallas as pl
from jax.experimental.pallas import tpu as pltpu
from jax.experimental.pallas import tpu_sc as plsc

i32 = jnp.int32
f32 = jnp.float32

B = 64
NQ = 5000
NCLS = 80
N = NQ * NCLS
K = 300
KOUT = 320
CAP = 2048
CAP16 = CAP + 16
NBINS = 2048
CHUNK = 8000
NCHUNK = N // CHUNK

_mesh = plsc.VectorSubcoreMesh(core_axis_name="c", subcore_axis_name="s")


def _monokey(v):
    b = plsc.bitcast(v, i32)
    return b ^ ((b >> 31) & 0x7FFFFFFF)


@functools.partial(
    pl.kernel,
    mesh=_mesh,
    compiler_params=pltpu.CompilerParams(needs_layout_passes=False),
    out_type=(
        jax.ShapeDtypeStruct((B * KOUT,), i32),
        jax.ShapeDtypeStruct((B * 4 * KOUT,), f32),
        jax.ShapeDtypeStruct((B * KOUT,), f32),
    ),
    scratch_types=[
        pltpu.VMEM((CHUNK,), f32),
        pltpu.VMEM((CHUNK,), f32),
        pltpu.VMEM((16 * NBINS,), i32),
        pltpu.VMEM((CAP16,), f32),
        pltpu.VMEM((CAP16,), i32),
        pltpu.VMEM((CAP,), i32),
        pltpu.VMEM((CAP,), i32),
        pltpu.VMEM((CAP,), i32),
        pltpu.VMEM((CAP,), i32),
        pltpu.VMEM((32,), i32),
        pltpu.VMEM((32,), i32),
        pltpu.VMEM((4 * NQ,), f32),
        pltpu.VMEM((KOUT,), f32),
        pltpu.VMEM((KOUT,), i32),
        pltpu.VMEM((4 * KOUT,), f32),
        pltpu.VMEM((144,), f32),
        pltpu.SemaphoreType.DMA,
        pltpu.SemaphoreType.DMA,
        pltpu.SemaphoreType.DMA,
    ],
)
def _sc_topk(logits_hbm, boxes_hbm, sizes_hbm, labels_o, boxes_o, scores_o,
             ch0, ch1, hist, cval, cidx, skA, siA, skB, siB, rhist, roff,
             boxrow, sbuf, lbuf, obox, sizes_v, sem0, sem1, semb):
    wid = lax.axis_index("s") * 2 + lax.axis_index("c")
    iota = lax.iota(i32, 16)
    ones = jnp.ones((16,), i32)
    zeros16 = jnp.zeros((16,), i32)
    lane_base = iota * NBINS

    pltpu.sync_copy(sizes_hbm, sizes_v.at[pl.ds(0, 2 * B)])

    for r_i in range(2):
        r = wid * 2 + r_i

        @plsc.parallel_loop(0, NBINS, unroll=8)
        def zero_hist(t):
            hist[pl.ds(t * 16, 16)] = zeros16

        pltpu.async_copy(logits_hbm.at[pl.ds(r * N, CHUNK)], ch0, sem0)
        pltpu.async_copy(logits_hbm.at[pl.ds(r * N + CHUNK, CHUNK)], ch1, sem1)

        def hist_vreg(buf):
            @plsc.parallel_loop(0, CHUNK // 16, unroll=16)
            def body(j):
                v = buf[pl.ds(j * 16, 16)]
                key = _monokey(v)
                bins = (key >> 21) + 1024
                plsc.addupdate_scatter(hist, [lane_base + (bins ^ iota)], ones)

        def p1_pair(i, _):
            c0 = 2 * i
            pltpu.make_async_copy(
                logits_hbm.at[pl.ds(r * N, CHUNK)], ch0, sem0).wait()
            hist_vreg(ch0)

            @pl.when(c0 + 2 < NCHUNK)
            def _():
                pltpu.async_copy(
                    logits_hbm.at[pl.ds(r * N + (c0 + 2) * CHUNK, CHUNK)], ch0, sem0)

            pltpu.make_async_copy(
                logits_hbm.at[pl.ds(r * N, CHUNK)], ch1, sem1).wait()
            hist_vreg(ch1)

            @pl.when(c0 + 3 < NCHUNK)
            def _():
                pltpu.async_copy(
                    logits_hbm.at[pl.ds(r * N + (c0 + 3) * CHUNK, CHUNK)], ch1, sem1)

            return 0

        lax.fori_loop(0, NCHUNK // 2, p1_pair, 0)

        def scan_block(t, carry):
            total, bstar = carry
            blk = 127 - t

            comb = hist[pl.ds(blk * 16, 16)]
            for l in range(1, 16):
                comb = comb + jnp.take(hist[pl.ds(l * NBINS + blk * 16, 16)],
                                       iota ^ l)
            rc = plsc.cumsum(lax.rev(comb, (0,)))
            cum = rc + total
            bins_rev = blk * 16 + 15 - iota
            val = jnp.where(cum >= K + 1, bins_rev, -1)
            bstar = jnp.maximum(bstar, jnp.max(val))
            total = total + rc[15]
            return total, bstar

        _, bstar = lax.fori_loop(0, NBINS // 16, scan_block, (0, -1))
        thresh = (bstar - 1024) * 2097152

        @plsc.parallel_loop(0, CAP16 // 16, unroll=4)
        def init_cand(t):
            cval[pl.ds(t * 16, 16)] = jnp.full((16,), -1e30, f32)
            cidx[pl.ds(t * 16, 16)] = (t * 16 + iota) * 80

        pltpu.async_copy(logits_hbm.at[pl.ds(r * N, CHUNK)], ch0, sem0)
        pltpu.async_copy(logits_hbm.at[pl.ds(r * N + CHUNK, CHUNK)], ch1, sem1)
        pltpu.async_copy(boxes_hbm.at[pl.ds(r * 4 * NQ, 4 * NQ)], boxrow, semb)

        def collect_vreg(buf, cbase, offv):
            @plsc.parallel_loop(0, CHUNK // 16, unroll=8, carry=offv)
            def body(j, offv):
                v = buf[pl.ds(j * 16, 16)]
                key = _monokey(v)
                m = key >= thresh
                cs = plsc.cumsum(m.astype(i32))
                pc = plsc.all_reduce_population_count(m)
                slots = offv + cs - 1
                m2 = m & (slots < CAP)
                plsc.store_scatter(cval, [slots], v, mask=m2)
                plsc.store_scatter(cidx, [slots], cbase + j * 16 + iota, mask=m2)
                return offv + pc

            return body

        def p2_pair(i, offv):
            c0 = 2 * i
            pltpu.make_async_copy(
                logits_hbm.at[pl.ds(r * N, CHUNK)], ch0, sem0).wait()
            offv = collect_vreg(ch0, c0 * CHUNK, offv)

            @pl.when(c0 + 2 < NCHUNK)
            def _():
                pltpu.async_copy(
                    logits_hbm.at[pl.ds(r * N + (c0 + 2) * CHUNK, CHUNK)], ch0, sem0)

            pltpu.make_async_copy(
                logits_hbm.at[pl.ds(r * N, CHUNK)], ch1, sem1).wait()
            offv = collect_vreg(ch1, (c0 + 1) * CHUNK, offv)

            @pl.when(c0 + 3 < NCHUNK)
            def _():
                pltpu.async_copy(
                    logits_hbm.at[pl.ds(r * N + (c0 + 3) * CHUNK, CHUNK)], ch1, sem1)

            return offv

        lax.fori_loop(0, NCHUNK // 2, p2_pair, zeros16)

        @plsc.parallel_loop(0, CAP // 16, unroll=4)
        def score_vreg(t):
            v = cval[pl.ds(t * 16, 16)]
            s = 1.0 / (1.0 + jnp.exp(-v))
            skA[pl.ds(t * 16, 16)] = ~plsc.bitcast(s, i32)
            siA[pl.ds(t * 16, 16)] = cidx[pl.ds(t * 16, 16)]

        bufs = [(skA, siA), (skB, siB)]
        for p in range(7):
            src_k, src_i = bufs[p % 2]
            dst_k, dst_i = bufs[(p + 1) % 2]
            shift = 5 * p

            rhist[pl.ds(0, 16)] = zeros16
            rhist[pl.ds(16, 16)] = zeros16

            def rhist_body(t, _, src_k=src_k, shift=shift):
                d = (src_k[pl.ds(t * 16, 16)] >> shift) & 31
                occ, last = plsc.scan_count(d)
                plsc.addupdate_scatter(rhist, [d], occ, mask=last)
                return 0

            lax.fori_loop(0, CAP // 16, rhist_body, 0)

            h0 = rhist[pl.ds(0, 16)]
            h1 = rhist[pl.ds(16, 16)]
            c0 = plsc.cumsum(h0)
            c1 = plsc.cumsum(h1)
            roff[pl.ds(0, 16)] = c0 - h0
            roff[pl.ds(16, 16)] = c1 - h1 + c0[15]

            def perm_body(t, _, src_k=src_k, src_i=src_i,
                          dst_k=dst_k, dst_i=dst_i, shift=shift):
                k = src_k[pl.ds(t * 16, 16)]
                ii = src_i[pl.ds(t * 16, 16)]
                d = (k >> shift) & 31
                occ, last = plsc.scan_count(d)
                basev = plsc.load_gather(roff, [d])
                slot = basev + occ - 1
                plsc.store_scatter(dst_k, [slot], k)
                plsc.store_scatter(dst_i, [slot], ii)
                plsc.addupdate_scatter(roff, [d], occ, mask=last)
                return 0

            lax.fori_loop(0, CAP // 16, perm_body, 0)

        res_k, res_i = bufs[7 % 2]

        pltpu.make_async_copy(boxes_hbm.at[pl.ds(r * 4 * NQ, 4 * NQ)], boxrow, semb).wait()
        sz = sizes_v[pl.ds(2 * r, 16)]
        sx = sz[0]
        sy = sz[1]

        def emit(t, _):
            kk = res_k[pl.ds(t * 16, 16)]
            ii = res_i[pl.ds(t * 16, 16)]
            sbuf[pl.ds(t * 16, 16)] = plsc.bitcast(~kk, f32)
            q = ((ii.astype(f32) + 0.5) * (1.0 / 80.0)).astype(i32)
            lbuf[pl.ds(t * 16, 16)] = ii - q * 80
            q4 = q * 4
            cx = plsc.load_gather(boxrow, [q4])
            cy = plsc.load_gather(boxrow, [q4 + 1])
            w = plsc.load_gather(boxrow, [q4 + 2])
            h = plsc.load_gather(boxrow, [q4 + 3])
            hw = 0.5 * w
            hh = 0.5 * h
            slot4 = (t * 16 + iota) * 4
            plsc.store_scatter(obox, [slot4], (cx - hw) * sx)
            plsc.store_scatter(obox, [slot4 + 1], (cy - hh) * sy)
            plsc.store_scatter(obox, [slot4 + 2], (cx + hw) * sx)
            plsc.store_scatter(obox, [slot4 + 3], (cy + hh) * sy)
            return 0

        lax.fori_loop(0, KOUT // 16, emit, 0)

        pltpu.sync_copy(sbuf, scores_o.at[pl.ds(r * KOUT, KOUT)])
        pltpu.sync_copy(lbuf, labels_o.at[pl.ds(r * KOUT, KOUT)])
        pltpu.sync_copy(obox, boxes_o.at[pl.ds(r * 4 * KOUT, 4 * KOUT)])


def kernel(pred_logits, pred_boxes, orig_target_sizes):
    logits = pred_logits.reshape(B * N)
    boxes = pred_boxes.reshape(B * 4 * NQ)
    sizes = orig_target_sizes.reshape(2 * B)
    labels, boxes_out, scores = _sc_topk(logits, boxes, sizes)
    return (
        labels.reshape(B, KOUT)[:, :K],
        boxes_out.reshape(B, KOUT, 4)[:, :K],
        scores.reshape(B, KOUT)[:, :K],
    )


# speedup vs baseline: 1.0297x; 1.0297x over previous
import functools

import jax
import jax.numpy as jnp
from jax import lax
from jax.experimental import pallas as pl
from jax.experimental.pallas import tpu as pltpu
from jax.experimental.pallas import tpu_sc as plsc

i32 = jnp.int32
f32 = jnp.float32

B = 64
NQ = 5000
NCLS = 80
N = NQ * NCLS
K = 300
KOUT = 320
CAP = 2048
CAP16 = CAP + 16
NBINS = 2048
CHUNK = 8000
NCHUNK = N // CHUNK

_mesh = plsc.VectorSubcoreMesh(core_axis_name="c", subcore_axis_name="s")


def _monokey(v):
    b = plsc.bitcast(v, i32)
    return b ^ ((b >> 31) & 0x7FFFFFFF)


@functools.partial(
    pl.kernel,
    mesh=_mesh,
    compiler_params=pltpu.CompilerParams(needs_layout_passes=False),
    out_type=(
        jax.ShapeDtypeStruct((B * KOUT,), i32),
        jax.ShapeDtypeStruct((B * 4 * KOUT,), f32),
        jax.ShapeDtypeStruct((B * KOUT,), f32),
    ),
    scratch_types=[
        pltpu.VMEM((CHUNK,), f32),
        pltpu.VMEM((CHUNK,), f32),
        pltpu.VMEM((16 * NBINS,), i32),
        pltpu.VMEM((CAP16,), f32),
        pltpu.VMEM((CAP16,), i32),
        pltpu.VMEM((CAP,), i32),
        pltpu.VMEM((CAP,), i32),
        pltpu.VMEM((CAP,), i32),
        pltpu.VMEM((CAP,), i32),
        pltpu.VMEM((32,), i32),
        pltpu.VMEM((32,), i32),
        pltpu.VMEM((4 * NQ,), f32),
        pltpu.VMEM((KOUT,), f32),
        pltpu.VMEM((KOUT,), i32),
        pltpu.VMEM((4 * KOUT,), f32),
        pltpu.VMEM((144,), f32),
        pltpu.SemaphoreType.DMA,
        pltpu.SemaphoreType.DMA,
        pltpu.SemaphoreType.DMA,
    ],
)
def _sc_topk(logits_hbm, boxes_hbm, sizes_hbm, labels_o, boxes_o, scores_o,
             ch0, ch1, hist, cval, cidx, skA, siA, skB, siB, rhist, roff,
             boxrow, sbuf, lbuf, obox, sizes_v, sem0, sem1, semb):
    wid = lax.axis_index("s") * 2 + lax.axis_index("c")
    iota = lax.iota(i32, 16)
    ones = jnp.ones((16,), i32)
    zeros16 = jnp.zeros((16,), i32)
    lane_base = iota * NBINS

    pltpu.sync_copy(sizes_hbm, sizes_v.at[pl.ds(0, 2 * B)])

    for r_i in range(2):
        r = wid * 2 + r_i

        @plsc.parallel_loop(0, NBINS, unroll=8)
        def zero_hist(t):
            hist[pl.ds(t * 16, 16)] = zeros16

        pltpu.async_copy(logits_hbm.at[pl.ds(r * N, CHUNK)], ch0, sem0)
        pltpu.async_copy(logits_hbm.at[pl.ds(r * N + CHUNK, CHUNK)], ch1, sem1)

        def hist_vreg(buf):
            @plsc.parallel_loop(0, CHUNK // 16, unroll=8)
            def body(j):
                v = buf[pl.ds(j * 16, 16)]
                key = _monokey(v)
                bins = (key >> 21) + 1024
                plsc.addupdate_scatter(hist, [lane_base + (bins ^ iota)], ones)

        def p1_pair(i, _):
            c0 = 2 * i
            pltpu.make_async_copy(
                logits_hbm.at[pl.ds(r * N, CHUNK)], ch0, sem0).wait()
            hist_vreg(ch0)

            @pl.when(c0 + 2 < NCHUNK)
            def _():
                pltpu.async_copy(
                    logits_hbm.at[pl.ds(r * N + (c0 + 2) * CHUNK, CHUNK)], ch0, sem0)

            pltpu.make_async_copy(
                logits_hbm.at[pl.ds(r * N, CHUNK)], ch1, sem1).wait()
            hist_vreg(ch1)

            @pl.when(c0 + 3 < NCHUNK)
            def _():
                pltpu.async_copy(
                    logits_hbm.at[pl.ds(r * N + (c0 + 3) * CHUNK, CHUNK)], ch1, sem1)

            return 0

        lax.fori_loop(0, NCHUNK // 2, p1_pair, 0)

        def scan_block(t, carry):
            total, bstar = carry
            blk = 127 - t

            comb = hist[pl.ds(blk * 16, 16)]
            for l in range(1, 16):
                comb = comb + jnp.take(hist[pl.ds(l * NBINS + blk * 16, 16)],
                                       iota ^ l)
            rc = plsc.cumsum(lax.rev(comb, (0,)))
            cum = rc + total
            bins_rev = blk * 16 + 15 - iota
            val = jnp.where(cum >= K + 1, bins_rev, -1)
            bstar = jnp.maximum(bstar, jnp.max(val))
            total = total + rc[15]
            return total, bstar

        _, bstar = lax.fori_loop(0, NBINS // 16, scan_block, (0, -1))
        thresh = (bstar - 1024) * 2097152

        @plsc.parallel_loop(0, CAP16 // 16, unroll=4)
        def init_cand(t):
            cval[pl.ds(t * 16, 16)] = jnp.full((16,), -1e30, f32)
            cidx[pl.ds(t * 16, 16)] = (t * 16 + iota) * 80

        pltpu.async_copy(logits_hbm.at[pl.ds(r * N, CHUNK)], ch0, sem0)
        pltpu.async_copy(logits_hbm.at[pl.ds(r * N + CHUNK, CHUNK)], ch1, sem1)
        pltpu.async_copy(boxes_hbm.at[pl.ds(r * 4 * NQ, 4 * NQ)], boxrow, semb)

        def collect_vreg(buf, cbase, offv):
            @plsc.parallel_loop(0, CHUNK // 16, unroll=4, carry=offv)
            def body(j, offv):
                v = buf[pl.ds(j * 16, 16)]
                key = _monokey(v)
                m = key >= thresh
                cs = plsc.cumsum(m.astype(i32))
                pc = plsc.all_reduce_population_count(m)
                slots = offv + cs - 1
                m2 = m & (slots < CAP)
                plsc.store_scatter(cval, [slots], v, mask=m2)
                plsc.store_scatter(cidx, [slots], cbase + j * 16 + iota, mask=m2)
                return offv + pc

            return body

        def p2_pair(i, offv):
            c0 = 2 * i
            pltpu.make_async_copy(
                logits_hbm.at[pl.ds(r * N, CHUNK)], ch0, sem0).wait()
            offv = collect_vreg(ch0, c0 * CHUNK, offv)

            @pl.when(c0 + 2 < NCHUNK)
            def _():
                pltpu.async_copy(
                    logits_hbm.at[pl.ds(r * N + (c0 + 2) * CHUNK, CHUNK)], ch0, sem0)

            pltpu.make_async_copy(
                logits_hbm.at[pl.ds(r * N, CHUNK)], ch1, sem1).wait()
            offv = collect_vreg(ch1, (c0 + 1) * CHUNK, offv)

            @pl.when(c0 + 3 < NCHUNK)
            def _():
                pltpu.async_copy(
                    logits_hbm.at[pl.ds(r * N + (c0 + 3) * CHUNK, CHUNK)], ch1, sem1)

            return offv

        lax.fori_loop(0, NCHUNK // 2, p2_pair, zeros16)

        @plsc.parallel_loop(0, CAP // 16, unroll=4)
        def score_vreg(t):
            v = cval[pl.ds(t * 16, 16)]
            s = 1.0 / (1.0 + jnp.exp(-v))
            skA[pl.ds(t * 16, 16)] = ~plsc.bitcast(s, i32)
            siA[pl.ds(t * 16, 16)] = cidx[pl.ds(t * 16, 16)]

        bufs = [(skA, siA), (skB, siB)]
        for p in range(7):
            src_k, src_i = bufs[p % 2]
            dst_k, dst_i = bufs[(p + 1) % 2]
            shift = 5 * p

            rhist[pl.ds(0, 16)] = zeros16
            rhist[pl.ds(16, 16)] = zeros16

            def rhist_body(t, _, src_k=src_k, shift=shift):
                d = (src_k[pl.ds(t * 16, 16)] >> shift) & 31
                occ, last = plsc.scan_count(d)
                plsc.addupdate_scatter(rhist, [d], occ, mask=last)
                return 0

            lax.fori_loop(0, CAP // 16, rhist_body, 0)

            h0 = rhist[pl.ds(0, 16)]
            h1 = rhist[pl.ds(16, 16)]
            c0 = plsc.cumsum(h0)
            c1 = plsc.cumsum(h1)
            roff[pl.ds(0, 16)] = c0 - h0
            roff[pl.ds(16, 16)] = c1 - h1 + c0[15]

            def perm_body(t, _, src_k=src_k, src_i=src_i,
                          dst_k=dst_k, dst_i=dst_i, shift=shift):
                k = src_k[pl.ds(t * 16, 16)]
                ii = src_i[pl.ds(t * 16, 16)]
                d = (k >> shift) & 31
                occ, last = plsc.scan_count(d)
                basev = plsc.load_gather(roff, [d])
                slot = basev + occ - 1
                plsc.store_scatter(dst_k, [slot], k)
                plsc.store_scatter(dst_i, [slot], ii)
                plsc.addupdate_scatter(roff, [d], occ, mask=last)
                return 0

            lax.fori_loop(0, CAP // 16, perm_body, 0)

        res_k, res_i = bufs[7 % 2]

        pltpu.make_async_copy(boxes_hbm.at[pl.ds(r * 4 * NQ, 4 * NQ)], boxrow, semb).wait()
        sz = sizes_v[pl.ds(2 * r, 16)]
        sx = sz[0]
        sy = sz[1]

        def emit(t, _):
            kk = res_k[pl.ds(t * 16, 16)]
            ii = res_i[pl.ds(t * 16, 16)]
            sbuf[pl.ds(t * 16, 16)] = plsc.bitcast(~kk, f32)
            q = ((ii.astype(f32) + 0.5) * (1.0 / 80.0)).astype(i32)
            lbuf[pl.ds(t * 16, 16)] = ii - q * 80
            q4 = q * 4
            cx = plsc.load_gather(boxrow, [q4])
            cy = plsc.load_gather(boxrow, [q4 + 1])
            w = plsc.load_gather(boxrow, [q4 + 2])
            h = plsc.load_gather(boxrow, [q4 + 3])
            hw = 0.5 * w
            hh = 0.5 * h
            slot4 = (t * 16 + iota) * 4
            plsc.store_scatter(obox, [slot4], (cx - hw) * sx)
            plsc.store_scatter(obox, [slot4 + 1], (cy - hh) * sy)
            plsc.store_scatter(obox, [slot4 + 2], (cx + hw) * sx)
            plsc.store_scatter(obox, [slot4 + 3], (cy + hh) * sy)
            return 0

        lax.fori_loop(0, KOUT // 16, emit, 0)

        pltpu.sync_copy(sbuf, scores_o.at[pl.ds(r * KOUT, KOUT)])
        pltpu.sync_copy(lbuf, labels_o.at[pl.ds(r * KOUT, KOUT)])
        pltpu.sync_copy(obox, boxes_o.at[pl.ds(r * 4 * KOUT, 4 * KOUT)])


def kernel(pred_logits, pred_boxes, orig_target_sizes):
    logits = pred_logits.reshape(B * N)
    boxes = pred_boxes.reshape(B * 4 * NQ)
    sizes = orig_target_sizes.reshape(2 * B)
    labels, boxes_out, scores = _sc_topk(logits, boxes, sizes)
    return (
        labels.reshape(B, KOUT)[:, :K],
        boxes_out.reshape(B, KOUT, 4)[:, :K],
        scores.reshape(B, KOUT)[:, :K],
    )
